# baseline (device time: 46861 ns/iter reference)
import jax
import jax.numpy as jnp
from jax import lax
from jax.experimental import pallas as pl
from jax.experimental.pallas import tpu as pltpu

P = 288


def kernel(x, assign, W1, W2):
    t, d = x.shape
    e_loc, _, f = W1.shape
    n_exp = 2 * e_loc
    bp = e_loc * P

    assign2d = assign.reshape(t, 1)

    def body(x_ref, a_ref, w1_ref, w2_ref, out_ref,
             xg, xcat0, xrecv1, yl, ysend, yrecv, w1f, w2f, w1b, w2b,
             wsems, send_sems, recv_sems):
        px = lax.axis_index("x")
        peer = (1 - px, lax.axis_index("y"), lax.axis_index("z"))

        def w_dma(le):
            return (pltpu.make_async_copy(w1_ref.at[le], w1f, wsems.at[0]),
                    pltpu.make_async_copy(w2_ref.at[le], w2f, wsems.at[1]))

        def y_rdma(le):
            return pltpu.make_async_remote_copy(
                src_ref=ysend.at[pl.ds(le * P, P)],
                dst_ref=yrecv.at[pl.ds(le * P, P)],
                send_sem=send_sems.at[2 * le + 1],
                recv_sem=recv_sems.at[2 * le + 1],
                device_id=peer, device_id_type=pl.DeviceIdType.MESH)

        dma_w0 = w_dma(0)
        for c in dma_w0:
            c.start()

        barrier_sem = pltpu.get_barrier_semaphore()
        pl.semaphore_signal(barrier_sem, inc=1, device_id=peer,
                            device_id_type=pl.DeviceIdType.MESH)
        pl.semaphore_wait(barrier_sem, 1)

        a = a_ref[...]
        cols4 = lax.broadcasted_iota(jnp.int32, (t, n_exp), 1)
        ohb = (a == cols4).astype(jnp.bfloat16)
        tri = (lax.broadcasted_iota(jnp.int32, (t, t), 0) >
               lax.broadcasted_iota(jnp.int32, (t, t), 1)).astype(jnp.bfloat16)
        rank = jnp.dot(tri, ohb, preferred_element_type=jnp.float32)
        rank_own = jnp.sum(ohb.astype(jnp.float32) * rank, axis=1,
                           keepdims=True).astype(jnp.int32)
        slot = lax.rem(a + 2 * px, n_exp)
        dcol = slot * P + rank_own

        xb = x_ref[...].astype(jnp.bfloat16)
        colsP = lax.broadcasted_iota(jnp.int32, (t, P), 1)

        def gathP(off):
            M = (dcol == colsP + off).astype(jnp.bfloat16)
            return lax.dot_general(
                M, xb, dimension_numbers=(((0,), (0,)), ((), ())),
                preferred_element_type=jnp.float32).astype(jnp.bfloat16)

        xg[0:P, :] = gathP(bp)
        rdma_x0 = pltpu.make_async_remote_copy(
            src_ref=xg.at[pl.ds(0, P)], dst_ref=xcat0.at[pl.ds(P, P)],
            send_sem=send_sems.at[0], recv_sem=recv_sems.at[0],
            device_id=peer, device_id_type=pl.DeviceIdType.MESH)
        rdma_x0.start()
        xg[P:, :] = gathP(bp + P)
        rdma_x1 = pltpu.make_async_remote_copy(
            src_ref=xg.at[pl.ds(P, P)], dst_ref=xrecv1,
            send_sem=send_sems.at[2], recv_sem=recv_sems.at[2],
            device_id=peer, device_id_type=pl.DeviceIdType.MESH)
        rdma_x1.start()

        cols2 = lax.broadcasted_iota(jnp.int32, (t, bp), 1)
        Ml = (dcol == cols2).astype(jnp.bfloat16)
        xcat0[0:P, :] = lax.dot_general(
            Ml[:, 0:P], xb, dimension_numbers=(((0,), (0,)), ((), ())),
            preferred_element_type=jnp.float32).astype(jnp.bfloat16)
        xgl1 = lax.dot_general(
            Ml[:, P:], xb, dimension_numbers=(((0,), (0,)), ((), ())),
            preferred_element_type=jnp.float32).astype(jnp.bfloat16)
        StA = Ml
        StB0 = (dcol == colsP + bp).astype(jnp.bfloat16)
        StB1 = (dcol == colsP + bp + P).astype(jnp.bfloat16)

        def ffn(xv):
            h = jnp.dot(xv, w1b[...], preferred_element_type=jnp.float32)
            hb = jnp.maximum(h, 0.0).astype(jnp.bfloat16)
            return jnp.dot(hb, w2b[...], preferred_element_type=jnp.float32)

        for c in dma_w0:
            c.wait()
        w1b[...] = w1f[...].astype(jnp.bfloat16)
        w2b[...] = w2f[...].astype(jnp.bfloat16)
        dma_w1 = w_dma(1)
        for c in dma_w1:
            c.start()
        rdma_x0.wait()
        ycat0 = ffn(xcat0[...])
        yl[0:P, :] = ycat0[0:P, :].astype(jnp.bfloat16)
        ysend[0:P, :] = ycat0[P:, :].astype(jnp.bfloat16)
        rdma_y0 = y_rdma(0)
        rdma_y0.start()

        for c in dma_w1:
            c.wait()
        w1b[...] = w1f[...].astype(jnp.bfloat16)
        w2b[...] = w2f[...].astype(jnp.bfloat16)
        rdma_x1.wait()
        ysend[P:, :] = ffn(xrecv1[...]).astype(jnp.bfloat16)
        rdma_y1 = y_rdma(1)
        rdma_y1.start()
        yl[P:, :] = ffn(xgl1).astype(jnp.bfloat16)

        acc = jnp.dot(StA, yl[...], preferred_element_type=jnp.float32)
        rdma_y0.wait()
        acc = acc + jnp.dot(StB0, yrecv[0:P, :],
                            preferred_element_type=jnp.float32)
        rdma_y1.wait()
        out_ref[...] = acc + jnp.dot(StB1, yrecv[P:, :],
                                     preferred_element_type=jnp.float32)

    out = pl.pallas_call(
        body,
        out_shape=jax.ShapeDtypeStruct((t, d), jnp.float32),
        in_specs=[
            pl.BlockSpec(memory_space=pltpu.VMEM),
            pl.BlockSpec(memory_space=pltpu.VMEM),
            pl.BlockSpec(memory_space=pltpu.MemorySpace.HBM),
            pl.BlockSpec(memory_space=pltpu.MemorySpace.HBM),
        ],
        out_specs=pl.BlockSpec(memory_space=pltpu.VMEM),
        scratch_shapes=[
            pltpu.VMEM((bp, d), jnp.bfloat16),
            pltpu.VMEM((bp, d), jnp.bfloat16),
            pltpu.VMEM((P, d), jnp.bfloat16),
            pltpu.VMEM((bp, d), jnp.bfloat16),
            pltpu.VMEM((bp, d), jnp.bfloat16),
            pltpu.VMEM((bp, d), jnp.bfloat16),
            pltpu.VMEM((d, f), jnp.float32),
            pltpu.VMEM((f, d), jnp.float32),
            pltpu.VMEM((d, f), jnp.bfloat16),
            pltpu.VMEM((f, d), jnp.bfloat16),
            pltpu.SemaphoreType.DMA((2,)),
            pltpu.SemaphoreType.DMA((4,)),
            pltpu.SemaphoreType.DMA((4,)),
        ],
        compiler_params=pltpu.CompilerParams(
            collective_id=0,
            vmem_limit_bytes=100 * 1024 * 1024,
        ),
    )(x, assign2d, W1, W2)
    return out


# device time: 46425 ns/iter; 1.0094x vs baseline; 1.0094x over previous
import jax
import jax.numpy as jnp
from jax import lax
from jax.experimental import pallas as pl
from jax.experimental.pallas import tpu as pltpu

P = 288


def kernel(x, assign, W1, W2):
    t, d = x.shape
    e_loc, _, f = W1.shape
    n_exp = 2 * e_loc
    bp = e_loc * P

    assign2d = assign.reshape(t, 1)

    def body(x_ref, a_ref, w1_ref, w2_ref, out_ref,
             xg, xrecv, yl, ysend, yrecv, w1f, w2f, w1b, w2b,
             wsems, send_sems, recv_sems):
        px = lax.axis_index("x")
        peer = (1 - px, lax.axis_index("y"), lax.axis_index("z"))

        def w_dma(le):
            return (pltpu.make_async_copy(w1_ref.at[le], w1f, wsems.at[0]),
                    pltpu.make_async_copy(w2_ref.at[le], w2f, wsems.at[1]))

        def y_rdma(le):
            return pltpu.make_async_remote_copy(
                src_ref=ysend.at[pl.ds(le * P, P)],
                dst_ref=yrecv.at[pl.ds(le * P, P)],
                send_sem=send_sems.at[2 * le + 1],
                recv_sem=recv_sems.at[2 * le + 1],
                device_id=peer, device_id_type=pl.DeviceIdType.MESH)

        dma_w0 = w_dma(0)
        for c in dma_w0:
            c.start()

        barrier_sem = pltpu.get_barrier_semaphore()
        pl.semaphore_signal(barrier_sem, inc=1, device_id=peer,
                            device_id_type=pl.DeviceIdType.MESH)
        pl.semaphore_wait(barrier_sem, 1)

        a = a_ref[...]
        cols4 = lax.broadcasted_iota(jnp.int32, (t, n_exp), 1)
        ohb = (a == cols4).astype(jnp.bfloat16)
        tri = (lax.broadcasted_iota(jnp.int32, (t, t), 0) >
               lax.broadcasted_iota(jnp.int32, (t, t), 1)).astype(jnp.bfloat16)
        rank = jnp.dot(tri, ohb, preferred_element_type=jnp.float32)
        rank_own = jnp.sum(ohb.astype(jnp.float32) * rank, axis=1,
                           keepdims=True).astype(jnp.int32)
        slot = lax.rem(a + 2 * px, n_exp)
        dcol = slot * P + rank_own

        xb = x_ref[...].astype(jnp.bfloat16)
        colsP = lax.broadcasted_iota(jnp.int32, (t, P), 1)

        def gathP(off):
            M = (dcol == colsP + off).astype(jnp.bfloat16)
            return lax.dot_general(
                M, xb, dimension_numbers=(((0,), (0,)), ((), ())),
                preferred_element_type=jnp.float32).astype(jnp.bfloat16)

        rdma_x = []
        for le in range(e_loc):
            xg[le * P:(le + 1) * P, :] = gathP(bp + le * P)
            r = pltpu.make_async_remote_copy(
                src_ref=xg.at[pl.ds(le * P, P)],
                dst_ref=xrecv.at[pl.ds(le * P, P)],
                send_sem=send_sems.at[2 * le], recv_sem=recv_sems.at[2 * le],
                device_id=peer, device_id_type=pl.DeviceIdType.MESH)
            r.start()
            rdma_x.append(r)

        cols2 = lax.broadcasted_iota(jnp.int32, (t, bp), 1)
        Ml = (dcol == cols2).astype(jnp.bfloat16)
        xgl = lax.dot_general(
            Ml, xb, dimension_numbers=(((0,), (0,)), ((), ())),
            preferred_element_type=jnp.float32).astype(jnp.bfloat16)
        StA = Ml
        StB0 = (dcol == colsP + bp).astype(jnp.bfloat16)
        StB1 = (dcol == colsP + bp + P).astype(jnp.bfloat16)

        def ffn(xv):
            h = jnp.dot(xv, w1b[...], preferred_element_type=jnp.float32)
            hb = jnp.maximum(h, 0.0).astype(jnp.bfloat16)
            return jnp.dot(hb, w2b[...], preferred_element_type=jnp.float32)

        for c in dma_w0:
            c.wait()
        w1b[...] = w1f[...].astype(jnp.bfloat16)
        w2b[...] = w2f[...].astype(jnp.bfloat16)
        dma_w1 = w_dma(1)
        for c in dma_w1:
            c.start()
        yl[0:P, :] = ffn(xgl[0:P, :]).astype(jnp.bfloat16)

        rdma_x[0].wait()
        ysend[0:P, :] = ffn(xrecv[0:P, :]).astype(jnp.bfloat16)
        rdma_y0 = y_rdma(0)
        rdma_y0.start()

        for c in dma_w1:
            c.wait()
        w1b[...] = w1f[...].astype(jnp.bfloat16)
        w2b[...] = w2f[...].astype(jnp.bfloat16)
        rdma_x[1].wait()
        ysend[P:, :] = ffn(xrecv[P:, :]).astype(jnp.bfloat16)
        rdma_y1 = y_rdma(1)
        rdma_y1.start()
        yl[P:, :] = ffn(xgl[P:bp, :]).astype(jnp.bfloat16)

        acc = jnp.dot(StA, yl[...], preferred_element_type=jnp.float32)
        rdma_y0.wait()
        acc = acc + jnp.dot(StB0, yrecv[0:P, :],
                            preferred_element_type=jnp.float32)
        rdma_y1.wait()
        out_ref[...] = acc + jnp.dot(StB1, yrecv[P:, :],
                                     preferred_element_type=jnp.float32)

    out = pl.pallas_call(
        body,
        out_shape=jax.ShapeDtypeStruct((t, d), jnp.float32),
        in_specs=[
            pl.BlockSpec(memory_space=pltpu.VMEM),
            pl.BlockSpec(memory_space=pltpu.VMEM),
            pl.BlockSpec(memory_space=pltpu.MemorySpace.HBM),
            pl.BlockSpec(memory_space=pltpu.MemorySpace.HBM),
        ],
        out_specs=pl.BlockSpec(memory_space=pltpu.VMEM),
        scratch_shapes=[
            pltpu.VMEM((bp, d), jnp.bfloat16),
            pltpu.VMEM((bp, d), jnp.bfloat16),
            pltpu.VMEM((bp, d), jnp.bfloat16),
            pltpu.VMEM((bp, d), jnp.bfloat16),
            pltpu.VMEM((bp, d), jnp.bfloat16),
            pltpu.VMEM((d, f), jnp.float32),
            pltpu.VMEM((f, d), jnp.float32),
            pltpu.VMEM((d, f), jnp.bfloat16),
            pltpu.VMEM((f, d), jnp.bfloat16),
            pltpu.SemaphoreType.DMA((2,)),
            pltpu.SemaphoreType.DMA((4,)),
            pltpu.SemaphoreType.DMA((4,)),
        ],
        compiler_params=pltpu.CompilerParams(
            collective_id=0,
            vmem_limit_bytes=100 * 1024 * 1024,
        ),
    )(x, assign2d, W1, W2)
    return out


# device time: 46284 ns/iter; 1.0125x vs baseline; 1.0030x over previous
import jax
import jax.numpy as jnp
from jax import lax
from jax.experimental import pallas as pl
from jax.experimental.pallas import tpu as pltpu

P = 288


def kernel(x, assign, W1, W2):
    t, d = x.shape
    e_loc, _, f = W1.shape
    n_exp = 2 * e_loc
    bp = e_loc * P

    assign2d = assign.reshape(t, 1)

    def body(x_ref, a_ref, w1_ref, w2_ref, out_ref,
             xv, xg, xrecv, yl, ysend, yrecv, w1f, w2f, w1b, w2b,
             xsem, wsems, send_sems, recv_sems):
        px = lax.axis_index("x")
        peer = (1 - px, lax.axis_index("y"), lax.axis_index("z"))

        def w_dma(le):
            return (pltpu.make_async_copy(w1_ref.at[le], w1f, wsems.at[0]),
                    pltpu.make_async_copy(w2_ref.at[le], w2f, wsems.at[1]))

        dma_x = pltpu.make_async_copy(x_ref, xv, xsem)
        dma_x.start()

        def y_rdma(le):
            return pltpu.make_async_remote_copy(
                src_ref=ysend.at[pl.ds(le * P, P)],
                dst_ref=yrecv.at[pl.ds(le * P, P)],
                send_sem=send_sems.at[2 * le + 1],
                recv_sem=recv_sems.at[2 * le + 1],
                device_id=peer, device_id_type=pl.DeviceIdType.MESH)

        dma_w0 = w_dma(0)
        for c in dma_w0:
            c.start()

        barrier_sem = pltpu.get_barrier_semaphore()
        pl.semaphore_signal(barrier_sem, inc=1, device_id=peer,
                            device_id_type=pl.DeviceIdType.MESH)
        pl.semaphore_wait(barrier_sem, 1)

        a = a_ref[...]
        cols4 = lax.broadcasted_iota(jnp.int32, (t, n_exp), 1)
        ohb = (a == cols4).astype(jnp.bfloat16)
        tri = (lax.broadcasted_iota(jnp.int32, (t, t), 0) >
               lax.broadcasted_iota(jnp.int32, (t, t), 1)).astype(jnp.bfloat16)
        rank = jnp.dot(tri, ohb, preferred_element_type=jnp.float32)
        rank_own = jnp.sum(ohb.astype(jnp.float32) * rank, axis=1,
                           keepdims=True).astype(jnp.int32)
        slot = lax.rem(a + 2 * px, n_exp)
        dcol = slot * P + rank_own

        dma_x.wait()
        xb = xv[...].astype(jnp.bfloat16)
        colsP = lax.broadcasted_iota(jnp.int32, (t, P), 1)

        def gathP(off):
            M = (dcol == colsP + off).astype(jnp.bfloat16)
            return lax.dot_general(
                M, xb, dimension_numbers=(((0,), (0,)), ((), ())),
                preferred_element_type=jnp.float32).astype(jnp.bfloat16)

        rdma_x = []
        for le in range(e_loc):
            xg[le * P:(le + 1) * P, :] = gathP(bp + le * P)
            r = pltpu.make_async_remote_copy(
                src_ref=xg.at[pl.ds(le * P, P)],
                dst_ref=xrecv.at[pl.ds(le * P, P)],
                send_sem=send_sems.at[2 * le], recv_sem=recv_sems.at[2 * le],
                device_id=peer, device_id_type=pl.DeviceIdType.MESH)
            r.start()
            rdma_x.append(r)

        cols2 = lax.broadcasted_iota(jnp.int32, (t, bp), 1)
        Ml = (dcol == cols2).astype(jnp.bfloat16)
        xgl = lax.dot_general(
            Ml, xb, dimension_numbers=(((0,), (0,)), ((), ())),
            preferred_element_type=jnp.float32).astype(jnp.bfloat16)
        StA = Ml
        StB0 = (dcol == colsP + bp).astype(jnp.bfloat16)
        StB1 = (dcol == colsP + bp + P).astype(jnp.bfloat16)

        def ffn(xv):
            h = jnp.dot(xv, w1b[...], preferred_element_type=jnp.float32)
            hb = jnp.maximum(h, 0.0).astype(jnp.bfloat16)
            return jnp.dot(hb, w2b[...], preferred_element_type=jnp.float32)

        for c in dma_w0:
            c.wait()
        w1b[...] = w1f[...].astype(jnp.bfloat16)
        w2b[...] = w2f[...].astype(jnp.bfloat16)
        dma_w1 = w_dma(1)
        for c in dma_w1:
            c.start()
        yl[0:P, :] = ffn(xgl[0:P, :]).astype(jnp.bfloat16)

        rdma_x[0].wait()
        ysend[0:P, :] = ffn(xrecv[0:P, :]).astype(jnp.bfloat16)
        rdma_y0 = y_rdma(0)
        rdma_y0.start()

        for c in dma_w1:
            c.wait()
        w1b[...] = w1f[...].astype(jnp.bfloat16)
        w2b[...] = w2f[...].astype(jnp.bfloat16)
        rdma_x[1].wait()
        ysend[P:, :] = ffn(xrecv[P:, :]).astype(jnp.bfloat16)
        rdma_y1 = y_rdma(1)
        rdma_y1.start()
        yl[P:, :] = ffn(xgl[P:bp, :]).astype(jnp.bfloat16)

        acc = jnp.dot(StA, yl[...], preferred_element_type=jnp.float32)
        rdma_y0.wait()
        acc = acc + jnp.dot(StB0, yrecv[0:P, :],
                            preferred_element_type=jnp.float32)
        rdma_y1.wait()
        out_ref[...] = acc + jnp.dot(StB1, yrecv[P:, :],
                                     preferred_element_type=jnp.float32)

    out = pl.pallas_call(
        body,
        out_shape=jax.ShapeDtypeStruct((t, d), jnp.float32),
        in_specs=[
            pl.BlockSpec(memory_space=pltpu.MemorySpace.HBM),
            pl.BlockSpec(memory_space=pltpu.VMEM),
            pl.BlockSpec(memory_space=pltpu.MemorySpace.HBM),
            pl.BlockSpec(memory_space=pltpu.MemorySpace.HBM),
        ],
        out_specs=pl.BlockSpec(memory_space=pltpu.VMEM),
        scratch_shapes=[
            pltpu.VMEM((t, d), jnp.float32),
            pltpu.VMEM((bp, d), jnp.bfloat16),
            pltpu.VMEM((bp, d), jnp.bfloat16),
            pltpu.VMEM((bp, d), jnp.bfloat16),
            pltpu.VMEM((bp, d), jnp.bfloat16),
            pltpu.VMEM((bp, d), jnp.bfloat16),
            pltpu.VMEM((d, f), jnp.float32),
            pltpu.VMEM((f, d), jnp.float32),
            pltpu.VMEM((d, f), jnp.bfloat16),
            pltpu.VMEM((f, d), jnp.bfloat16),
            pltpu.SemaphoreType.DMA,
            pltpu.SemaphoreType.DMA((2,)),
            pltpu.SemaphoreType.DMA((4,)),
            pltpu.SemaphoreType.DMA((4,)),
        ],
        compiler_params=pltpu.CompilerParams(
            collective_id=0,
            vmem_limit_bytes=100 * 1024 * 1024,
        ),
    )(x, assign2d, W1, W2)
    return out
